# baseline (speedup 1.0000x reference)
import functools

import jax
import jax.numpy as jnp
from jax import lax
from jax.experimental import pallas as pl
from jax.experimental.pallas import tpu as pltpu
from jax.experimental.pallas import tpu_sc as plsc

VOCAB = 4096
D = 4096
B_SZ, T_SZ = 8, 2048
TOK = B_SZ * T_SZ
NC, NS, L = 2, 16, 16
NW = NC * NS
TPW = TOK // NW
TILES_PER_BATCH = T_SZ // TPW
K = 8
NCH = TPW // K

ROW_BLK = 1024
N_BLK = VOCAB // ROW_BLK


def _lse_body(emb_ref, out_ref):
    x = emb_ref[...]
    m = jnp.max(x, axis=1, keepdims=True)
    s = jnp.sum(jnp.exp(x - m), axis=1, keepdims=True)
    out_ref[...] = (m + jnp.log(s)).reshape(1, 1, ROW_BLK)


def _lse_table(emb):
    out = pl.pallas_call(
        _lse_body,
        grid=(N_BLK,),
        in_specs=[pl.BlockSpec((ROW_BLK, D), lambda i: (i, 0))],
        out_specs=pl.BlockSpec((1, 1, ROW_BLK), lambda i: (i, 0, 0)),
        out_shape=jax.ShapeDtypeStruct((N_BLK, 1, ROW_BLK), jnp.float32),
    )(emb)
    return out.reshape(VOCAB)


def _relay_body(emb_h, idx_h, tgt_h, lse_h, out_h, part_h,
                idx_v, tgt_v, lse_v, rows_v, acc_v,
                gsem0, gsem1, osem0, osem1):
    wid = lax.axis_index("s") * NC + lax.axis_index("c")
    b = wid // TILES_PER_BATCH
    base = (wid % TILES_PER_BATCH) * TPW
    pltpu.sync_copy(idx_h.at[pl.ds(wid * TPW, TPW)], idx_v)
    pltpu.sync_copy(tgt_h.at[pl.ds(wid * TPW, TPW)], tgt_v)
    pltpu.sync_copy(lse_h, lse_v)
    acc_v[...] = jnp.zeros((L,), jnp.float32)
    lanes = lax.iota(jnp.int32, L)
    gsems = (gsem0, gsem1)
    osems = (osem0, osem1)

    def gather_start(j, bf):
        pltpu.async_copy(
            emb_h.at[idx_v.at[pl.ds(j * K, K)]],
            rows_v.at[pl.ds(bf * K, K)], gsems[bf])

    def gather_wait(bf):
        pltpu.make_async_copy(
            emb_h.at[pl.ds(0, K)], rows_v.at[pl.ds(bf * K, K)],
            gsems[bf]).wait()

    def out_wait(bf):
        pltpu.make_async_copy(
            rows_v.at[pl.ds(bf * K, K)], out_h.at[b, pl.ds(base, K)],
            osems[bf]).wait()

    gather_start(0, 0)

    def pair(p, carry):
        for bf in (0, 1):
            j = 2 * p + bf
            nb = 1 - bf

            @pl.when(j + 1 < NCH)
            def _():
                @pl.when(j >= 1)
                def _():
                    out_wait(nb)
                gather_start(j + 1, nb)

            gather_wait(bf)
            pltpu.async_copy(
                rows_v.at[pl.ds(bf * K, K)],
                out_h.at[b, pl.ds(base + j * K, K)], osems[bf])
            idx16 = idx_v[pl.ds(p * L, L)]
            tgt16 = tgt_v[pl.ds(p * L, L)]
            rowid = bf * K + jnp.clip(lanes - bf * K, 0, K - 1)
            valid = (lanes >= bf * K) & (lanes < bf * K + K)
            picked = plsc.load_gather(rows_v, [rowid, tgt16])
            lses = plsc.load_gather(lse_v, [idx16])
            acc_v[...] = acc_v[...] + jnp.where(valid, lses - picked, 0.0)
        return carry

    lax.fori_loop(0, NCH // 2, pair, 0)
    out_wait(0)
    out_wait(1)
    pltpu.sync_copy(acc_v, part_h.at[pl.ds(wid * L, L)])


_relay = functools.partial(
    pl.kernel,
    mesh=plsc.VectorSubcoreMesh(core_axis_name="c", subcore_axis_name="s"),
    compiler_params=pltpu.CompilerParams(needs_layout_passes=False),
    out_type=[
        jax.ShapeDtypeStruct((B_SZ, T_SZ, D), jnp.float32),
        jax.ShapeDtypeStruct((NW * L,), jnp.float32),
    ],
    scratch_types=[
        pltpu.VMEM((TPW,), jnp.int32),
        pltpu.VMEM((TPW,), jnp.int32),
        pltpu.VMEM((VOCAB,), jnp.float32),
        pltpu.VMEM((2 * K, D), jnp.float32),
        pltpu.VMEM((L,), jnp.float32),
        pltpu.SemaphoreType.DMA,
        pltpu.SemaphoreType.DMA,
        pltpu.SemaphoreType.DMA,
        pltpu.SemaphoreType.DMA,
    ],
)(_relay_body)


def kernel(idx, targets, emb):
    idx_f = idx.reshape(-1).astype(jnp.int32)
    tgt_f = targets.reshape(-1).astype(jnp.int32)
    lse = _lse_table(emb)
    logits, part = _relay(emb, idx_f, tgt_f, lse)
    loss = jnp.sum(part) / TOK
    return logits, loss


# speedup vs baseline: 1.0041x; 1.0041x over previous
import functools

import jax
import jax.numpy as jnp
from jax import lax
from jax.experimental import pallas as pl
from jax.experimental.pallas import tpu as pltpu
from jax.experimental.pallas import tpu_sc as plsc

VOCAB = 4096
D = 4096
B_SZ, T_SZ = 8, 2048
TOK = B_SZ * T_SZ
NC, NS, L = 2, 16, 16
NW = NC * NS
TPW = TOK // NW
TILES_PER_BATCH = T_SZ // TPW
K = 8
NCH = TPW // K

ROW_BLK = 512
N_BLK = VOCAB // ROW_BLK


def _lse_body(emb_ref, out_ref):
    x = emb_ref[...]
    m = jnp.max(x, axis=1, keepdims=True)
    s = jnp.sum(jnp.exp(x - m), axis=1, keepdims=True)
    out_ref[...] = (m + jnp.log(s)).reshape(1, 1, ROW_BLK)


def _lse_table(emb):
    out = pl.pallas_call(
        _lse_body,
        grid=(N_BLK,),
        in_specs=[pl.BlockSpec((ROW_BLK, D), lambda i: (i, 0))],
        out_specs=pl.BlockSpec((1, 1, ROW_BLK), lambda i: (i, 0, 0)),
        out_shape=jax.ShapeDtypeStruct((N_BLK, 1, ROW_BLK), jnp.float32),
    )(emb)
    return out.reshape(VOCAB)


def _relay_body(emb_h, idx_h, tgt_h, lse_h, out_h, part_h,
                idx_v, tgt_v, lse_v, rows_v, acc_v,
                gsem0, gsem1, osem0, osem1):
    wid = lax.axis_index("s") * NC + lax.axis_index("c")
    b = wid // TILES_PER_BATCH
    base = (wid % TILES_PER_BATCH) * TPW
    pltpu.sync_copy(idx_h.at[pl.ds(wid * TPW, TPW)], idx_v)
    pltpu.sync_copy(tgt_h.at[pl.ds(wid * TPW, TPW)], tgt_v)
    pltpu.sync_copy(lse_h, lse_v)
    acc_v[...] = jnp.zeros((L,), jnp.float32)
    lanes = lax.iota(jnp.int32, L)
    gsems = (gsem0, gsem1)
    osems = (osem0, osem1)

    def gather_start(j, bf):
        pltpu.async_copy(
            emb_h.at[idx_v.at[pl.ds(j * K, K)]],
            rows_v.at[pl.ds(bf * K, K)], gsems[bf])

    def gather_wait(bf):
        pltpu.make_async_copy(
            emb_h.at[pl.ds(0, K)], rows_v.at[pl.ds(bf * K, K)],
            gsems[bf]).wait()

    def out_wait(bf):
        pltpu.make_async_copy(
            rows_v.at[pl.ds(bf * K, K)], out_h.at[b, pl.ds(base, K)],
            osems[bf]).wait()

    gather_start(0, 0)

    def pair(p, carry):
        for bf in (0, 1):
            j = 2 * p + bf
            nb = 1 - bf

            @pl.when(j + 1 < NCH)
            def _():
                @pl.when(j >= 1)
                def _():
                    out_wait(nb)
                gather_start(j + 1, nb)

            gather_wait(bf)
            pltpu.async_copy(
                rows_v.at[pl.ds(bf * K, K)],
                out_h.at[b, pl.ds(base + j * K, K)], osems[bf])
            idx16 = idx_v[pl.ds(p * L, L)]
            tgt16 = tgt_v[pl.ds(p * L, L)]
            rowid = bf * K + jnp.clip(lanes - bf * K, 0, K - 1)
            valid = (lanes >= bf * K) & (lanes < bf * K + K)
            picked = plsc.load_gather(rows_v, [rowid, tgt16])
            lses = plsc.load_gather(lse_v, [idx16])
            acc_v[...] = acc_v[...] + jnp.where(valid, lses - picked, 0.0)
        return carry

    lax.fori_loop(0, NCH // 2, pair, 0)
    out_wait(0)
    out_wait(1)
    pltpu.sync_copy(acc_v, part_h.at[pl.ds(wid * L, L)])


_relay = functools.partial(
    pl.kernel,
    mesh=plsc.VectorSubcoreMesh(core_axis_name="c", subcore_axis_name="s"),
    compiler_params=pltpu.CompilerParams(needs_layout_passes=False),
    out_type=[
        jax.ShapeDtypeStruct((B_SZ, T_SZ, D), jnp.float32),
        jax.ShapeDtypeStruct((NW * L,), jnp.float32),
    ],
    scratch_types=[
        pltpu.VMEM((TPW,), jnp.int32),
        pltpu.VMEM((TPW,), jnp.int32),
        pltpu.VMEM((VOCAB,), jnp.float32),
        pltpu.VMEM((2 * K, D), jnp.float32),
        pltpu.VMEM((L,), jnp.float32),
        pltpu.SemaphoreType.DMA,
        pltpu.SemaphoreType.DMA,
        pltpu.SemaphoreType.DMA,
        pltpu.SemaphoreType.DMA,
    ],
)(_relay_body)


def kernel(idx, targets, emb):
    idx_f = idx.reshape(-1).astype(jnp.int32)
    tgt_f = targets.reshape(-1).astype(jnp.int32)
    lse = _lse_table(emb)
    logits, part = _relay(emb, idx_f, tgt_f, lse)
    loss = jnp.sum(part) / TOK
    return logits, loss
